# baseline (speedup 1.0000x reference)
import functools

import jax
import jax.numpy as jnp
from jax import lax
from jax.experimental import pallas as pl
from jax.experimental.pallas import tpu as pltpu
from jax.experimental.pallas import tpu_sc as plsc

N_NODES = 10000
N_EDGES = 320000
D_FEAT = 128
NC = 2
NS = 16
DH = D_FEAT // NC
NGRP = DH // 16
N_PAD = 10112
ROWS_PER_SUB = N_PAD // NS
E_PER_SUB = N_EDGES // NS
BLK = 79
NBLK = ROWS_PER_SUB // BLK
CHUNK = 80
CH_PER_SUB = E_PER_SUB // CHUNK
NBUF = 5
NCH = 25
NGROUP = NCH // NBUF
N_IBLK = CH_PER_SUB // NCH


def _sc_body(tab, rowi2d, col2d, tr2d, out, t1, T, A, b0, b1,
             rows0, rows1, rows2, rows3, rows4,
             rblk, cblk, tvb,
             g0s, g1s, g2s, g3s, g4s, s0s, s1s, s2s, s3s, s4s):
    rows = [rows0, rows1, rows2, rows3, rows4]
    gsem = [g0s, g1s, g2s, g3s, g4s]
    ssem = [s0s, s1s, s2s, s3s, s4s]
    c = lax.axis_index("c")
    s = lax.axis_index("s")
    r0 = s * ROWS_PER_SUB
    g0 = c * N_PAD + r0

    def zrow(r, _):
        for j in range(NGRP):
            b1[r, pl.ds(16 * j, 16)] = jnp.zeros((16,), jnp.float32)
        return _
    lax.fori_loop(0, BLK, zrow, None)
    for k in range(NBLK):
        pltpu.sync_copy(b1, T.at[pl.ds(r0 + k * BLK, BLK)])
        pltpu.sync_copy(b1, A.at[pl.ds(r0 + k * BLK, BLK)])
    plsc.subcore_barrier()

    def layer(src, acc):
        def scale_chunk(jj, buf):
            def scale(e, _):
                t16 = plsc.load_gather(
                    tvb, [jnp.full((16,), jj, jnp.int32),
                          jnp.full((16,), e, jnp.int32)])
                for j in range(NGRP):
                    d = pl.ds(16 * j, 16)
                    buf[e, d] = buf[e, d] * t16
                return _
            lax.fori_loop(0, CHUNK, scale, None, unroll=4)

        def iblk_body(b, _):
            ch0 = s * CH_PER_SUB + b * NCH
            pltpu.sync_copy(rowi2d.at[pl.ds(c * (N_EDGES // CHUNK) + ch0, NCH)],
                            rblk)
            pltpu.sync_copy(col2d.at[pl.ds(ch0, NCH)], cblk)
            pltpu.sync_copy(tr2d.at[pl.ds(ch0, NCH)], tvb)
            for k in range(NBUF):
                pltpu.async_copy(src.at[rblk.at[k]], rows[k], gsem[k])
            for g in range(NGROUP):
                for k in range(NBUF):
                    j = g * NBUF + k
                    pltpu.make_async_copy(src.at[rblk.at[j]], rows[k],
                                          gsem[k]).wait()
                    scale_chunk(j, rows[k])
                    pltpu.async_copy(rows[k], acc.at[cblk.at[j]], ssem[k],
                                     add=True)
                for k in range(NBUF):
                    j = g * NBUF + k
                    pltpu.make_async_copy(rows[k], acc.at[cblk.at[j]],
                                          ssem[k]).wait()
                    if g + 1 < NGROUP:
                        pltpu.async_copy(src.at[rblk.at[j + NBUF]], rows[k],
                                         gsem[k])
            return _
        lax.fori_loop(0, N_IBLK, iblk_body, None)

    layer(tab, A)
    plsc.subcore_barrier()
    for k in range(NBLK):
        pltpu.sync_copy(A.at[pl.ds(r0 + k * BLK, BLK)], b0)
        pltpu.sync_copy(b0, t1.at[pl.ds(g0 + k * BLK, BLK)])
    plsc.subcore_barrier()
    layer(t1, T)
    plsc.subcore_barrier()

    third = jnp.full((16,), 1.0 / 3.0, jnp.float32)
    def add1(r, _):
        for j in range(NGRP):
            d = pl.ds(16 * j, 16)
            b0[r, d] = b0[r, d] + b1[r, d]
        return _
    def add2(r, _):
        for j in range(NGRP):
            d = pl.ds(16 * j, 16)
            b0[r, d] = (b0[r, d] + b1[r, d]) * third
        return _
    for k in range(NBLK):
        pltpu.sync_copy(tab.at[pl.ds(g0 + k * BLK, BLK)], b0)
        pltpu.sync_copy(A.at[pl.ds(r0 + k * BLK, BLK)], b1)
        lax.fori_loop(0, BLK, add1, None, unroll=4)
        pltpu.sync_copy(T.at[pl.ds(r0 + k * BLK, BLK)], b1)
        lax.fori_loop(0, BLK, add2, None, unroll=4)
        pltpu.sync_copy(b0, out.at[pl.ds(g0 + k * BLK, BLK)])


_sc_kernel = functools.partial(
    pl.kernel,
    out_type=jax.ShapeDtypeStruct((NC * N_PAD, DH), jnp.float32),
    mesh=plsc.VectorSubcoreMesh(core_axis_name="c", subcore_axis_name="s"),
    compiler_params=pltpu.CompilerParams(
        needs_layout_passes=False, use_tc_tiling_on_sc=False),
    scratch_types=[
        pltpu.HBM((NC * N_PAD, DH), jnp.float32),
        pltpu.VMEM_SHARED((N_PAD, DH), jnp.float32),
        pltpu.VMEM_SHARED((N_PAD, DH), jnp.float32),
        pltpu.VMEM((BLK, DH), jnp.float32),
        pltpu.VMEM((BLK, DH), jnp.float32),
        pltpu.VMEM((CHUNK, DH), jnp.float32),
        pltpu.VMEM((CHUNK, DH), jnp.float32),
        pltpu.VMEM((CHUNK, DH), jnp.float32),
        pltpu.VMEM((CHUNK, DH), jnp.float32),
        pltpu.VMEM((CHUNK, DH), jnp.float32),
        pltpu.VMEM((NCH, CHUNK), jnp.int32),
        pltpu.VMEM((NCH, CHUNK), jnp.int32),
        pltpu.VMEM((NCH, CHUNK), jnp.float32),
        pltpu.SemaphoreType.DMA,
        pltpu.SemaphoreType.DMA,
        pltpu.SemaphoreType.DMA,
        pltpu.SemaphoreType.DMA,
        pltpu.SemaphoreType.DMA,
        pltpu.SemaphoreType.DMA,
        pltpu.SemaphoreType.DMA,
        pltpu.SemaphoreType.DMA,
        pltpu.SemaphoreType.DMA,
        pltpu.SemaphoreType.DMA,
    ],
)(_sc_body)


def kernel(embed, edge_index, trend):
    row = edge_index[0].astype(jnp.int32)
    col = edge_index[1].astype(jnp.int32)
    e_pad = jnp.pad(embed, ((0, N_PAD - N_NODES), (0, 0)))
    tab = e_pad.reshape(N_PAD, NC, DH).transpose(1, 0, 2).reshape(NC * N_PAD, DH)
    rowi2d = jnp.concatenate([row, row + N_PAD]).reshape(
        NC * N_EDGES // CHUNK, CHUNK)
    col2d = col.reshape(N_EDGES // CHUNK, CHUNK)
    tr2d = trend.astype(jnp.float32).reshape(N_EDGES // CHUNK, CHUNK)
    out = _sc_kernel(tab, rowi2d, col2d, tr2d)
    out = out.reshape(NC, N_PAD, DH).transpose(1, 0, 2).reshape(N_PAD, D_FEAT)
    return out[:N_NODES]
